# Initial kernel scaffold; baseline (speedup 1.0000x reference)
import jax
import jax.numpy as jnp
from jax.experimental import pallas as pl


def kernel(node_features, edge_index, W1, b1, W2, b2):
    raise NotImplementedError("write your pallas kernel here")


# speedup vs baseline: 6.8348x; 6.8348x over previous
import jax
import jax.numpy as jnp
from jax import lax
from jax.experimental import pallas as pl
from jax.experimental.pallas import tpu as pltpu
from jax.experimental.pallas import tpu_sc as plsc

N_NODES = 10000
N_EDGES = 320000
D_HID = 128
N_CLASSES = 64

NC = 2
NS = 16
NW = NC * NS
EPW = N_EDGES // NW
CK = 128
NFULL = EPW // CK
TAIL = EPW - NFULL * CK
GR = 80
NG = N_NODES // GR
GPT = (NG + NS - 1) // NS


def _make_agg(d):
  mesh = plsc.VectorSubcoreMesh(core_axis_name="c", subcore_axis_name="s",
                                num_cores=NC, num_subcores=NS)

  def body(h_hbm, src_hbm, dst_hbm, out_hbm,
           src_v, dst_v, rows_v, src_t, dst_t, rows_t, zbuf, acc_sh, sem):
    cid = lax.axis_index("c")
    sid = lax.axis_index("s")
    wid = sid * NC + cid

    zero16 = jnp.zeros((16,), jnp.float32)

    def zrow(r, carry):
      for j in range(d // 16):
        zbuf[r, pl.ds(j * 16, 16)] = zero16
      return carry

    lax.fori_loop(0, GR, zrow, 0)
    for it in range(GPT):
      g = sid + it * NS

      @pl.when(g < NG)
      def _():
        pltpu.sync_copy(zbuf, acc_sh.at[pl.ds(g * GR, GR)])

    plsc.subcore_barrier()

    e_base = wid * EPW

    def step(i, carry):
      e0 = e_base + i * CK
      pltpu.sync_copy(src_hbm.at[pl.ds(e0, CK)], src_v)
      pltpu.sync_copy(dst_hbm.at[pl.ds(e0, CK)], dst_v)
      pltpu.async_copy(h_hbm.at[src_v], rows_v, sem).wait()
      pltpu.sync_copy(rows_v, acc_sh.at[dst_v], add=True)
      return carry

    lax.fori_loop(0, NFULL, step, 0)

    e0 = e_base + NFULL * CK
    pltpu.sync_copy(src_hbm.at[pl.ds(e0, TAIL)], src_t)
    pltpu.sync_copy(dst_hbm.at[pl.ds(e0, TAIL)], dst_t)
    pltpu.async_copy(h_hbm.at[src_t], rows_t, sem).wait()
    pltpu.sync_copy(rows_t, acc_sh.at[dst_t], add=True)

    plsc.subcore_barrier()
    for it in range(GPT):
      g = sid + it * NS

      @pl.when(g < NG)
      def _():
        pltpu.sync_copy(acc_sh.at[pl.ds(g * GR, GR)], zbuf)
        pltpu.sync_copy(zbuf, out_hbm.at[cid, pl.ds(g * GR, GR)])

  return pl.kernel(
      body,
      out_type=jax.ShapeDtypeStruct((NC, N_NODES, d), jnp.float32),
      mesh=mesh,
      compiler_params=pltpu.CompilerParams(use_tc_tiling_on_sc=(d % 128 == 0)),
      scratch_types=[
          pltpu.VMEM((CK,), jnp.int32),
          pltpu.VMEM((CK,), jnp.int32),
          pltpu.VMEM((CK, d), jnp.float32),
          pltpu.VMEM((TAIL,), jnp.int32),
          pltpu.VMEM((TAIL,), jnp.int32),
          pltpu.VMEM((TAIL, d), jnp.float32),
          pltpu.VMEM((GR, d), jnp.float32),
          pltpu.VMEM_SHARED((N_NODES, d), jnp.float32),
          pltpu.SemaphoreType.DMA,
      ],
  )


_AGG_HID = _make_agg(D_HID)
_AGG_CLS = _make_agg(N_CLASSES)


def _mm_bias(x_ref, w_ref, b_ref, o_ref):
  o_ref[...] = jnp.dot(x_ref[...], w_ref[...],
                       preferred_element_type=jnp.float32) + b_ref[...]


def _combine_mm_bias(p_ref, w_ref, b_ref, o_ref):
  x = jnp.maximum(p_ref[0] + p_ref[1], 0.0)
  o_ref[...] = jnp.dot(x, w_ref[...],
                       preferred_element_type=jnp.float32) + b_ref[...]


def _combine_relu(p_ref, o_ref):
  o_ref[...] = jnp.maximum(p_ref[0] + p_ref[1], 0.0)


def kernel(node_features, edge_index, W1, b1, W2, b2):
  x = node_features.astype(jnp.float32)
  ei = edge_index.astype(jnp.int32)
  src, dst = ei[0], ei[1]

  h1 = pl.pallas_call(
      _mm_bias,
      out_shape=jax.ShapeDtypeStruct((N_NODES, D_HID), jnp.float32),
  )(x, W1, b1.reshape(1, D_HID))

  p1 = _AGG_HID(h1, src, dst)

  h2 = pl.pallas_call(
      _combine_mm_bias,
      out_shape=jax.ShapeDtypeStruct((N_NODES, N_CLASSES), jnp.float32),
  )(p1, W2, b2.reshape(1, N_CLASSES))

  p2 = _AGG_CLS(h2, src, dst)

  out = pl.pallas_call(
      _combine_relu,
      out_shape=jax.ShapeDtypeStruct((N_NODES, N_CLASSES), jnp.float32),
  )(p2)
  return out
